# Initial kernel scaffold; baseline (speedup 1.0000x reference)
import jax
import jax.numpy as jnp
from jax.experimental import pallas as pl


def kernel(x, W):
    raise NotImplementedError("write your pallas kernel here")


# speedup vs baseline: 1.6841x; 1.6841x over previous
import functools

import jax
import jax.numpy as jnp
from jax import lax
from jax.experimental import pallas as pl
from jax.experimental.pallas import tpu as pltpu
from jax.experimental.pallas import tpu_sc as plsc

_CHUNK = 128


def _gather_body(n_ch, table_hbm, idx_hbm, out_hbm, idx_v, rows_v, gsem):
    nc = plsc.get_sparse_core_info().num_cores
    wid = lax.axis_index("s") * nc + lax.axis_index("c")
    pltpu.sync_copy(idx_hbm.at[wid], idx_v)

    def body(j, _):
        pltpu.async_copy(table_hbm.at[idx_v.at[j]], rows_v, gsem).wait()
        pltpu.sync_copy(rows_v, out_hbm.at[wid, j])
        return 0

    lax.fori_loop(0, n_ch, body, 0)


def kernel(x, W):
    Bt, S = x.shape
    V, D = W.shape
    B = Bt * S
    info = plsc.get_sparse_core_info()
    nw = info.num_cores * info.num_subcores
    assert B % (nw * _CHUNK) == 0
    n_ch = B // (nw * _CHUNK)

    idx = x.reshape(nw, n_ch, _CHUNK).astype(jnp.int32)

    mesh = plsc.VectorSubcoreMesh(core_axis_name="c", subcore_axis_name="s")
    k = pl.kernel(
        functools.partial(_gather_body, n_ch),
        out_type=jax.ShapeDtypeStruct((nw, n_ch, _CHUNK, D), jnp.float32),
        mesh=mesh,
        scratch_types=[
            pltpu.VMEM((n_ch, _CHUNK), jnp.int32),
            pltpu.VMEM((_CHUNK, D), jnp.float32),
            pltpu.SemaphoreType.DMA,
        ],
        compiler_params=pltpu.CompilerParams(use_tc_tiling_on_sc=False),
    )
    out = k(W, idx)
    return out.reshape(Bt, S, D)


# speedup vs baseline: 1.8691x; 1.1099x over previous
import functools

import jax
import jax.numpy as jnp
from jax import lax
from jax.experimental import pallas as pl
from jax.experimental.pallas import tpu as pltpu
from jax.experimental.pallas import tpu_sc as plsc

_CHUNK = 128


_NBUF = 4


def _gather_body(n_ch, table_hbm, idx_hbm, out_hbm, idx_v, rows_v,
                 gsems, ssems):
    nc = plsc.get_sparse_core_info().num_cores
    wid = lax.axis_index("s") * nc + lax.axis_index("c")
    pltpu.sync_copy(idx_hbm.at[wid], idx_v)

    def start_gather(b, j):
        pltpu.async_copy(table_hbm.at[idx_v.at[j]], rows_v.at[b], gsems[b])

    def wait_gather(b, j):
        pltpu.make_async_copy(table_hbm.at[idx_v.at[j]], rows_v.at[b],
                              gsems[b]).wait()

    def start_store(b, j):
        pltpu.async_copy(rows_v.at[b], out_hbm.at[wid, j], ssems[b])

    def wait_store(b, j):
        pltpu.make_async_copy(rows_v.at[b], out_hbm.at[wid, j],
                              ssems[b]).wait()

    for b in range(_NBUF):
        start_gather(b, b)

    n_outer = n_ch // _NBUF

    def outer(g, _):
        for b in range(_NBUF):
            j = g * _NBUF + b
            wait_gather(b, j)
            start_store(b, j)
        for b in range(_NBUF):
            j = g * _NBUF + b
            jn = j + _NBUF
            wait_store(b, j)

            @pl.when(jn < n_ch)
            def _():
                start_gather(b, jn)

        return 0

    lax.fori_loop(0, n_outer, outer, 0)


def kernel(x, W):
    Bt, S = x.shape
    V, D = W.shape
    B = Bt * S
    info = plsc.get_sparse_core_info()
    nw = info.num_cores * info.num_subcores
    assert B % (nw * _CHUNK) == 0
    n_ch = B // (nw * _CHUNK)

    idx = x.reshape(nw, n_ch, _CHUNK).astype(jnp.int32)

    mesh = plsc.VectorSubcoreMesh(core_axis_name="c", subcore_axis_name="s")
    k = pl.kernel(
        functools.partial(_gather_body, n_ch),
        out_type=jax.ShapeDtypeStruct((nw, n_ch, _CHUNK, D), jnp.float32),
        mesh=mesh,
        scratch_types=[
            pltpu.VMEM((n_ch, _CHUNK), jnp.int32),
            pltpu.VMEM((_NBUF, _CHUNK, D), jnp.float32),
            [pltpu.SemaphoreType.DMA] * _NBUF,
            [pltpu.SemaphoreType.DMA] * _NBUF,
        ],
        compiler_params=pltpu.CompilerParams(use_tc_tiling_on_sc=False),
    )
    out = k(W, idx)
    return out.reshape(Bt, S, D)


# speedup vs baseline: 1.8706x; 1.0008x over previous
import functools

import jax
import jax.numpy as jnp
from jax import lax
from jax.experimental import pallas as pl
from jax.experimental.pallas import tpu as pltpu
from jax.experimental.pallas import tpu_sc as plsc

_CHUNK = 128


_NBUF = 4


def _gather_body(n_ch, table_hbm, idx_hbm, out_hbm, idx_v, rows_v,
                 gsems, ssems):
    nc = plsc.get_sparse_core_info().num_cores
    wid = lax.axis_index("s") * nc + lax.axis_index("c")
    base = wid * n_ch * _CHUNK
    pltpu.sync_copy(idx_hbm.at[wid], idx_v)

    def start_gather(b, j):
        pltpu.async_copy(table_hbm.at[idx_v.at[j]], rows_v.at[b], gsems[b])

    def wait_gather(b, j):
        pltpu.make_async_copy(table_hbm.at[idx_v.at[j]], rows_v.at[b],
                              gsems[b]).wait()

    def start_store(b, j):
        pltpu.async_copy(rows_v.at[b], out_hbm.at[pl.ds(base + j * _CHUNK,
                                                        _CHUNK)], ssems[b])

    def wait_store(b, j):
        pltpu.make_async_copy(rows_v.at[b],
                              out_hbm.at[pl.ds(base + j * _CHUNK, _CHUNK)],
                              ssems[b]).wait()

    for b in range(_NBUF):
        start_gather(b, b)

    n_outer = n_ch // _NBUF

    def outer(g, _):
        for b in range(_NBUF):
            j = g * _NBUF + b
            wait_gather(b, j)
            start_store(b, j)
        for b in range(_NBUF):
            j = g * _NBUF + b
            jn = j + _NBUF
            wait_store(b, j)

            @pl.when(jn < n_ch)
            def _():
                start_gather(b, jn)

        return 0

    lax.fori_loop(0, n_outer, outer, 0)


def kernel(x, W):
    Bt, S = x.shape
    V, D = W.shape
    B = Bt * S
    info = plsc.get_sparse_core_info()
    nw = info.num_cores * info.num_subcores
    assert B % (nw * _CHUNK) == 0
    n_ch = B // (nw * _CHUNK)

    idx = x.reshape(nw, n_ch, _CHUNK).astype(jnp.int32)

    mesh = plsc.VectorSubcoreMesh(core_axis_name="c", subcore_axis_name="s")
    k = pl.kernel(
        functools.partial(_gather_body, n_ch),
        out_type=jax.ShapeDtypeStruct((B, D), jnp.float32),
        mesh=mesh,
        scratch_types=[
            pltpu.VMEM((n_ch, _CHUNK), jnp.int32),
            pltpu.VMEM((_NBUF, _CHUNK, D), jnp.float32),
            [pltpu.SemaphoreType.DMA] * _NBUF,
            [pltpu.SemaphoreType.DMA] * _NBUF,
        ],
        compiler_params=pltpu.CompilerParams(use_tc_tiling_on_sc=False),
    )
    out = k(W, idx)
    return out.reshape(Bt, S, D)
